# baseline (speedup 1.0000x reference)
import functools

import jax
import jax.numpy as jnp
from jax import lax
from jax.experimental import pallas as pl
from jax.experimental.pallas import tpu as pltpu
from jax.experimental.pallas import tpu_sc as plsc

NNB = 10
NC, NS = 2, 16
LANES = 16
NW = NC * NS


def kernel(msgs, ts, counts):
    N, L, D = msgs.shape
    NB = 80
    ROWS = NB * NNB
    NBLK = N // NB
    CHUNK = 80
    NCHUNK = ROWS // CHUNK
    ITERS = (NBLK + NW - 1) // NW

    msgs_flat = msgs.reshape(N * L, D)
    ts_flat = ts.reshape(N * L)

    mesh = plsc.VectorSubcoreMesh(
        core_axis_name="c", subcore_axis_name="s",
        num_cores=NC, num_subcores=NS)

    @functools.partial(
        pl.kernel,
        out_type=(
            jax.ShapeDtypeStruct((N * NNB, D), jnp.float32),
            jax.ShapeDtypeStruct((N * NNB,), jnp.int32),
        ),
        mesh=mesh,
        compiler_params=pltpu.CompilerParams(needs_layout_passes=False),
        scratch_types=[
            pltpu.VMEM((NB,), jnp.int32),
            pltpu.VMEM((NB * L,), jnp.int32),
            pltpu.VMEM((ROWS,), jnp.int32),
            pltpu.VMEM((ROWS,), jnp.int32),
            pltpu.VMEM((ROWS, D), jnp.float32),
            pltpu.SemaphoreType.DMA,
        ],
    )
    def sc_kernel(msgs_hbm, ts_hbm, counts_hbm, feats_out, ts_out,
                  counts_v, ts_v, idx_v, tso_v, stage, sem):
        wid = lax.axis_index("s") * NC + lax.axis_index("c")

        def block_body(i, carry):
            b = wid + i * NW

            @pl.when(b < NBLK)
            def _():
                node0 = b * NB
                pltpu.sync_copy(counts_hbm.at[pl.ds(node0, NB)], counts_v)
                pltpu.sync_copy(ts_hbm.at[pl.ds(node0 * L, NB * L)], ts_v)

                lane = lax.iota(jnp.int32, LANES)
                for g in range(NB // LANES):
                    nloc = g * LANES + lane
                    c = counts_v[pl.ds(g * LANES, LANES)]
                    for j in range(NNB):
                        idx = c - NNB + j
                        valid = idx >= 0
                        idx_cl = jnp.maximum(idx, 0)
                        grow = (node0 + nloc) * L + idx_cl
                        pos = nloc * NNB + j
                        plsc.store_scatter(idx_v, [pos], grow)
                        tsv = plsc.load_gather(ts_v, [nloc * L + idx_cl])
                        tsv = jnp.where(valid, tsv,
                                        jnp.full((LANES,), -1, jnp.int32))
                        plsc.store_scatter(tso_v, [pos], tsv)

                copies = []
                for k in range(NCHUNK):
                    copies.append(pltpu.async_copy(
                        msgs_hbm.at[idx_v.at[pl.ds(k * CHUNK, CHUNK)]],
                        stage.at[pl.ds(k * CHUNK, CHUNK)],
                        sem))
                for cp in copies:
                    cp.wait()

                zeros16 = jnp.zeros((LANES,), jnp.float32)

                for g in range(NB // LANES):
                    c16 = counts_v[pl.ds(g * LANES, LANES)]
                    for t in range(LANES):
                        z = jnp.maximum(0, NNB - c16[t])
                        n = g * LANES + t

                        def zero_row(j, carry3, n=n):
                            row = n * NNB + j
                            for v in range(D // LANES):
                                stage[row, pl.ds(v * LANES, LANES)] = zeros16
                            return carry3

                        lax.fori_loop(0, z, zero_row, jnp.int32(0))

                pltpu.sync_copy(stage, feats_out.at[pl.ds(b * ROWS, ROWS)])
                pltpu.sync_copy(tso_v, ts_out.at[pl.ds(b * ROWS, ROWS)])

            return carry

        lax.fori_loop(0, ITERS, block_body, jnp.int32(0))

    feats, ts_o = sc_kernel(msgs_flat, ts_flat, counts)
    return feats.reshape(N, NNB, D), ts_o.reshape(N, NNB)
